# baseline (speedup 1.0000x reference)
import functools

import jax
import jax.numpy as jnp
from jax import lax
from jax.experimental import pallas as pl
from jax.experimental.pallas import tpu as pltpu
from jax.experimental.pallas import tpu_sc as plsc

B, S = 1024, 50
N = B * S
HIDDEN = 1024
EPS = 1e-05

NC, NS = 2, 16
NW = NC * NS
TOK_PER_W = N // NW
CHUNK = 64
NCHUNK = TOK_PER_W // CHUNK
BIG_WIDTHS = (256, 128, 128, 128)

TOK_TILE = 512
GRID = N // TOK_TILE


def _sc_gather(ids_list, sp_t, mv_t, it_t, ab_t):
    mesh = plsc.VectorSubcoreMesh(core_axis_name="c", subcore_axis_name="s")

    @functools.partial(
        pl.kernel,
        mesh=mesh,
        out_type=[jax.ShapeDtypeStruct((N, w), jnp.float32) for w in BIG_WIDTHS],
        scratch_types=[
            pltpu.VMEM((TOK_PER_W,), jnp.int32),
            pltpu.VMEM((TOK_PER_W,), jnp.int32),
            pltpu.VMEM((TOK_PER_W,), jnp.int32),
            pltpu.VMEM((TOK_PER_W,), jnp.int32),
            pltpu.VMEM((CHUNK, 256), jnp.float32),
            pltpu.VMEM((CHUNK, 128), jnp.float32),
            pltpu.VMEM((CHUNK, 128), jnp.float32),
            pltpu.VMEM((CHUNK, 128), jnp.float32),
            pltpu.SemaphoreType.DMA,
            pltpu.SemaphoreType.DMA,
        ],
    )
    def k(sp_ids, mv_ids, it_ids, ab_ids, sp_hbm, mv_hbm, it_hbm, ab_hbm,
          o_sp, o_mv, o_it, o_ab,
          i_sp, i_mv, i_it, i_ab, b_sp, b_mv, b_it, b_ab, gsem, ssem):
        wid = lax.axis_index("s") * NC + lax.axis_index("c")
        base_w = pl.multiple_of(wid * TOK_PER_W, TOK_PER_W)
        idxs = (i_sp, i_mv, i_it, i_ab)
        for t, ids_hbm in enumerate((sp_ids, mv_ids, it_ids, ab_ids)):
            pltpu.sync_copy(ids_hbm.at[wid], idxs[t])
        tables = (sp_hbm, mv_hbm, it_hbm, ab_hbm)
        bufs = (b_sp, b_mv, b_it, b_ab)
        outs = (o_sp, o_mv, o_it, o_ab)

        def body(kk, carry):
            off = pl.multiple_of(kk * CHUNK, CHUNK)
            gathers = [
                pltpu.async_copy(
                    tables[t].at[idxs[t].at[pl.ds(off, CHUNK)]], bufs[t], gsem)
                for t in range(4)
            ]
            stores = []
            for t in range(4):
                gathers[t].wait()
                stores.append(pltpu.async_copy(
                    bufs[t], outs[t].at[pl.ds(base_w + off, CHUNK)], ssem))
            for st in stores:
                st.wait()
            return carry

        lax.fori_loop(0, NCHUNK, body, 0)

    return k(*ids_list, sp_t, mv_t, it_t, ab_t)


def _tc_body(st_ids_ref, we_ids_ref, te_ids_ref, po_ids_ref,
             sp_ref, mv_ref, it_ref, ab_ref, hp_ref, bo_ref,
             st_t_ref, we_t_ref, te_t_ref, po_t_ref,
             hp_W_ref, hp_b_ref, boost_W_ref, boost_b_ref,
             wproj_ref, proj_b_ref, gamma_ref, beta_ref, out_ref):
    f32 = jnp.float32

    def onehot_emb(ids_ref, tbl_ref, vocab):
        ids = ids_ref[0, 0, :]
        oh = (ids[:, None] == lax.broadcasted_iota(
            jnp.int32, (TOK_TILE, vocab), 1)).astype(f32)
        return jnp.dot(oh, tbl_ref[...], preferred_element_type=f32)

    st_emb = onehot_emb(st_ids_ref, st_t_ref, 8)
    we_emb = onehot_emb(we_ids_ref, we_t_ref, 10)
    te_emb = onehot_emb(te_ids_ref, te_t_ref, 5)
    po_emb = onehot_emb(po_ids_ref, po_t_ref, 12)
    hp_emb = hp_ref[...] * hp_W_ref[...] + hp_b_ref[...][None, :]
    bo_emb = jnp.dot(bo_ref[...], boost_W_ref[...],
                     preferred_element_type=f32) + boost_b_ref[...][None, :]

    bf16 = jnp.bfloat16
    combined = jnp.concatenate([
        sp_ref[...], mv_ref[...], it_ref[:, :64], ab_ref[:, :64],
        hp_emb, bo_emb, st_emb, we_emb, te_emb, po_emb], axis=1).astype(bf16)

    acc = jnp.dot(combined, wproj_ref[...], preferred_element_type=f32)
    acc = acc + proj_b_ref[...][None, :]
    mean = jnp.mean(acc, axis=1, keepdims=True)
    cen = acc - mean
    var = jnp.mean(cen * cen, axis=1, keepdims=True)
    y = cen * lax.rsqrt(var + EPS)
    out_ref[...] = y * gamma_ref[...][None, :] + beta_ref[...][None, :]


def _full(shape):
    nd = len(shape)
    return pl.BlockSpec(shape, lambda i: (0,) * nd)


def kernel(species_ids, move_ids, item_ids, ability_ids, hp_values, stat_boosts,
           status_ids, weather_ids, terrain_ids, position_ids,
           species_table, move_table, item_table, ability_table,
           hp_W, hp_b, boost_W, boost_b,
           status_table, weather_table, terrain_table, position_table,
           proj_W, proj_b, ln_gamma, ln_beta):
    i32 = jnp.int32
    ids_list = [a.reshape(NW, TOK_PER_W).astype(i32)
                for a in (species_ids, move_ids, item_ids, ability_ids)]

    pad64 = lambda t: jnp.pad(t, ((0, 0), (0, 64)))
    sp_e, mv_e, it_e, ab_e = _sc_gather(
        ids_list, species_table, move_table, pad64(item_table),
        pad64(ability_table))

    st3 = status_ids.reshape(GRID, 1, TOK_TILE).astype(i32)
    we3 = weather_ids.reshape(GRID, 1, TOK_TILE).astype(i32)
    te3 = terrain_ids.reshape(GRID, 1, TOK_TILE).astype(i32)
    po3 = position_ids.reshape(GRID, 1, TOK_TILE).astype(i32)
    hp2 = hp_values.reshape(N, 1)
    bo2 = stat_boosts.reshape(N, 7)
    wproj_bf = proj_W.astype(jnp.bfloat16)

    ids_spec = pl.BlockSpec((1, 1, TOK_TILE), lambda i: (i, 0, 0))

    out = pl.pallas_call(
        _tc_body,
        grid=(GRID,),
        in_specs=[
            ids_spec, ids_spec, ids_spec, ids_spec,
            pl.BlockSpec((TOK_TILE, 256), lambda i: (i, 0)),
            pl.BlockSpec((TOK_TILE, 128), lambda i: (i, 0)),
            pl.BlockSpec((TOK_TILE, 128), lambda i: (i, 0)),
            pl.BlockSpec((TOK_TILE, 128), lambda i: (i, 0)),
            pl.BlockSpec((TOK_TILE, 1), lambda i: (i, 0)),
            pl.BlockSpec((TOK_TILE, 7), lambda i: (i, 0)),
            _full((8, 32)), _full((10, 32)), _full((5, 32)), _full((12, 64)),
            _full((1, 32)), _full((32,)), _full((7, 32)), _full((32,)),
            _full((736, HIDDEN)), _full((HIDDEN,)),
            _full((HIDDEN,)), _full((HIDDEN,)),
        ],
        out_specs=pl.BlockSpec((TOK_TILE, HIDDEN), lambda i: (i, 0)),
        out_shape=jax.ShapeDtypeStruct((N, HIDDEN), jnp.float32),
        compiler_params=pltpu.CompilerParams(
            dimension_semantics=("arbitrary",)),
    )(st3, we3, te3, po3, sp_e, mv_e, it_e, ab_e, hp2, bo2,
      status_table, weather_table, terrain_table, position_table,
      hp_W, hp_b, boost_W, boost_b,
      wproj_bf, proj_b, ln_gamma, ln_beta)

    return out.reshape(B, S, HIDDEN)


# speedup vs baseline: 1.8578x; 1.8578x over previous
import functools

import jax
import jax.numpy as jnp
from jax import lax
from jax.experimental import pallas as pl
from jax.experimental.pallas import tpu as pltpu
from jax.experimental.pallas import tpu_sc as plsc

B, S = 1024, 50
N = B * S
HIDDEN = 1024
EPS = 1e-05

NC, NS = 2, 16
NW = NC * NS
TOK_PER_W = N // NW
CHUNK = 64
NCHUNK = TOK_PER_W // CHUNK
BIG_WIDTHS = (256, 128, 128, 128)

TOK_TILE = 512
GRID = N // TOK_TILE


def _sc_gather(ids_list, sp_t, mv_t, it_t, ab_t):
    mesh = plsc.VectorSubcoreMesh(core_axis_name="c", subcore_axis_name="s")

    @functools.partial(
        pl.kernel,
        mesh=mesh,
        out_type=[jax.ShapeDtypeStruct((N, w), jnp.float32) for w in BIG_WIDTHS],
        scratch_types=[
            pltpu.VMEM((TOK_PER_W,), jnp.int32),
            pltpu.VMEM((TOK_PER_W,), jnp.int32),
            pltpu.VMEM((TOK_PER_W,), jnp.int32),
            pltpu.VMEM((TOK_PER_W,), jnp.int32),
            pltpu.VMEM((CHUNK, 256), jnp.float32),
            pltpu.VMEM((CHUNK, 128), jnp.float32),
            pltpu.VMEM((CHUNK, 128), jnp.float32),
            pltpu.VMEM((CHUNK, 128), jnp.float32),
            pltpu.SemaphoreType.DMA,
            pltpu.SemaphoreType.DMA,
        ],
    )
    def k(sp_ids, mv_ids, it_ids, ab_ids, sp_hbm, mv_hbm, it_hbm, ab_hbm,
          o_sp, o_mv, o_it, o_ab,
          i_sp, i_mv, i_it, i_ab, b_sp, b_mv, b_it, b_ab, gsem, ssem):
        wid = lax.axis_index("s") * NC + lax.axis_index("c")
        base_w = pl.multiple_of(wid * TOK_PER_W, TOK_PER_W)
        idxs = (i_sp, i_mv, i_it, i_ab)
        for t, ids_hbm in enumerate((sp_ids, mv_ids, it_ids, ab_ids)):
            pltpu.sync_copy(ids_hbm.at[wid], idxs[t])
        tables = (sp_hbm, mv_hbm, it_hbm, ab_hbm)
        bufs = (b_sp, b_mv, b_it, b_ab)
        outs = (o_sp, o_mv, o_it, o_ab)

        def body(kk, carry):
            off = pl.multiple_of(kk * CHUNK, CHUNK)
            gathers = [
                pltpu.async_copy(
                    tables[t].at[idxs[t].at[pl.ds(off, CHUNK)]], bufs[t], gsem)
                for t in range(4)
            ]
            stores = []
            for t in range(4):
                gathers[t].wait()
                stores.append(pltpu.async_copy(
                    bufs[t], outs[t].at[pl.ds(base_w + off, CHUNK)], ssem))
            for st in stores:
                st.wait()
            return carry

        lax.fori_loop(0, NCHUNK, body, 0)

    return k(*ids_list, sp_t, mv_t, it_t, ab_t)


def _tc_body(st_ids_ref, we_ids_ref, te_ids_ref, po_ids_ref,
             sp_ref, mv_ref, it_ref, ab_ref, hp_ref, bo_ref,
             st_t_ref, we_t_ref, te_t_ref, po_t_ref,
             hp_W_ref, hp_b_ref, boost_W_ref, boost_b_ref,
             wproj_ref, proj_b_ref, gamma_ref, beta_ref, out_ref):
    f32 = jnp.float32

    def onehot_emb(ids_ref, tbl_ref, vocab):
        ids = ids_ref[0, 0, :]
        oh = (ids[:, None] == lax.broadcasted_iota(
            jnp.int32, (TOK_TILE, vocab), 1)).astype(f32)
        return jnp.dot(oh, tbl_ref[...], preferred_element_type=f32)

    st_emb = onehot_emb(st_ids_ref, st_t_ref, 8)
    we_emb = onehot_emb(we_ids_ref, we_t_ref, 10)
    te_emb = onehot_emb(te_ids_ref, te_t_ref, 5)
    po_emb = onehot_emb(po_ids_ref, po_t_ref, 12)
    hp_emb = hp_ref[...] * hp_W_ref[...] + hp_b_ref[...][None, :]
    bo_emb = jnp.dot(bo_ref[...], boost_W_ref[...],
                     preferred_element_type=f32) + boost_b_ref[...][None, :]

    bf16 = jnp.bfloat16
    combined = jnp.concatenate([
        sp_ref[...], mv_ref[...], it_ref[:, :64], ab_ref[:, :64],
        hp_emb, bo_emb, st_emb, we_emb, te_emb, po_emb], axis=1).astype(bf16)

    acc = jnp.dot(combined, wproj_ref[...], preferred_element_type=f32)
    acc = acc + proj_b_ref[...][None, :]
    mean = jnp.mean(acc, axis=1, keepdims=True)
    cen = acc - mean
    var = jnp.mean(cen * cen, axis=1, keepdims=True)
    y = cen * lax.rsqrt(var + EPS)
    out_ref[...] = y * gamma_ref[...][None, :] + beta_ref[...][None, :]


def _full(shape):
    nd = len(shape)
    return pl.BlockSpec(shape, lambda i: (0,) * nd)


def kernel(species_ids, move_ids, item_ids, ability_ids, hp_values, stat_boosts,
           status_ids, weather_ids, terrain_ids, position_ids,
           species_table, move_table, item_table, ability_table,
           hp_W, hp_b, boost_W, boost_b,
           status_table, weather_table, terrain_table, position_table,
           proj_W, proj_b, ln_gamma, ln_beta):
    i32 = jnp.int32
    ids_list = [a.T.reshape(NW, TOK_PER_W).astype(i32)
                for a in (species_ids, move_ids, item_ids, ability_ids)]

    pad64 = lambda t: jnp.pad(t, ((0, 0), (0, 64)))
    sp_e, mv_e, it_e, ab_e = _sc_gather(
        ids_list, species_table, move_table, pad64(item_table),
        pad64(ability_table))

    st3 = status_ids.T.reshape(GRID, 1, TOK_TILE).astype(i32)
    we3 = weather_ids.T.reshape(GRID, 1, TOK_TILE).astype(i32)
    te3 = terrain_ids.T.reshape(GRID, 1, TOK_TILE).astype(i32)
    po3 = position_ids.T.reshape(GRID, 1, TOK_TILE).astype(i32)
    hp2 = hp_values.T.reshape(N, 1)
    bo2 = stat_boosts.transpose(1, 0, 2).reshape(N, 7)
    wproj_bf = proj_W.astype(jnp.bfloat16)

    ids_spec = pl.BlockSpec((1, 1, TOK_TILE), lambda i: (i, 0, 0))

    out = pl.pallas_call(
        _tc_body,
        grid=(GRID,),
        in_specs=[
            ids_spec, ids_spec, ids_spec, ids_spec,
            pl.BlockSpec((TOK_TILE, 256), lambda i: (i, 0)),
            pl.BlockSpec((TOK_TILE, 128), lambda i: (i, 0)),
            pl.BlockSpec((TOK_TILE, 128), lambda i: (i, 0)),
            pl.BlockSpec((TOK_TILE, 128), lambda i: (i, 0)),
            pl.BlockSpec((TOK_TILE, 1), lambda i: (i, 0)),
            pl.BlockSpec((TOK_TILE, 7), lambda i: (i, 0)),
            _full((8, 32)), _full((10, 32)), _full((5, 32)), _full((12, 64)),
            _full((1, 32)), _full((32,)), _full((7, 32)), _full((32,)),
            _full((736, HIDDEN)), _full((HIDDEN,)),
            _full((HIDDEN,)), _full((HIDDEN,)),
        ],
        out_specs=pl.BlockSpec((TOK_TILE, HIDDEN), lambda i: (i, 0)),
        out_shape=jax.ShapeDtypeStruct((N, HIDDEN), jnp.float32),
        compiler_params=pltpu.CompilerParams(
            dimension_semantics=("arbitrary",)),
    )(st3, we3, te3, po3, sp_e, mv_e, it_e, ab_e, hp2, bo2,
      status_table, weather_table, terrain_table, position_table,
      hp_W, hp_b, boost_W, boost_b,
      wproj_bf, proj_b, ln_gamma, ln_beta)

    return out.reshape(S, B, HIDDEN).transpose(1, 0, 2)


# speedup vs baseline: 2.0177x; 1.0860x over previous
import functools

import jax
import jax.numpy as jnp
from jax import lax
from jax.experimental import pallas as pl
from jax.experimental.pallas import tpu as pltpu
from jax.experimental.pallas import tpu_sc as plsc

B, S = 1024, 50
N = B * S
HIDDEN = 1024
EPS = 1e-05

NC, NS = 2, 16
NW = NC * NS
TOK_PER_W = N // NW
CHUNK = 64
NCHUNK = TOK_PER_W // CHUNK
BIG_WIDTHS = (256, 128, 128, 128)

TOK_TILE = 1024
GRID = N // TOK_TILE


def _sc_gather(ids_list, sp_t, mv_t, it_t, ab_t):
    mesh = plsc.VectorSubcoreMesh(core_axis_name="c", subcore_axis_name="s")

    @functools.partial(
        pl.kernel,
        mesh=mesh,
        out_type=[jax.ShapeDtypeStruct((N, w), jnp.float32) for w in BIG_WIDTHS],
        scratch_types=[
            pltpu.VMEM((TOK_PER_W,), jnp.int32),
            pltpu.VMEM((TOK_PER_W,), jnp.int32),
            pltpu.VMEM((TOK_PER_W,), jnp.int32),
            pltpu.VMEM((TOK_PER_W,), jnp.int32),
            pltpu.VMEM((CHUNK, 256), jnp.float32),
            pltpu.VMEM((CHUNK, 128), jnp.float32),
            pltpu.VMEM((CHUNK, 128), jnp.float32),
            pltpu.VMEM((CHUNK, 128), jnp.float32),
            pltpu.SemaphoreType.DMA,
            pltpu.SemaphoreType.DMA,
        ],
    )
    def k(sp_ids, mv_ids, it_ids, ab_ids, sp_hbm, mv_hbm, it_hbm, ab_hbm,
          o_sp, o_mv, o_it, o_ab,
          i_sp, i_mv, i_it, i_ab, b_sp, b_mv, b_it, b_ab, gsem, ssem):
        wid = lax.axis_index("s") * NC + lax.axis_index("c")
        base_w = pl.multiple_of(wid * TOK_PER_W, TOK_PER_W)
        idxs = (i_sp, i_mv, i_it, i_ab)
        for t, ids_hbm in enumerate((sp_ids, mv_ids, it_ids, ab_ids)):
            pltpu.sync_copy(ids_hbm.at[wid], idxs[t])
        tables = (sp_hbm, mv_hbm, it_hbm, ab_hbm)
        bufs = (b_sp, b_mv, b_it, b_ab)
        outs = (o_sp, o_mv, o_it, o_ab)

        def body(kk, carry):
            off = pl.multiple_of(kk * CHUNK, CHUNK)
            gathers = [
                pltpu.async_copy(
                    tables[t].at[idxs[t].at[pl.ds(off, CHUNK)]], bufs[t], gsem)
                for t in range(4)
            ]
            stores = []
            for t in range(4):
                gathers[t].wait()
                stores.append(pltpu.async_copy(
                    bufs[t], outs[t].at[pl.ds(base_w + off, CHUNK)], ssem))
            for st in stores:
                st.wait()
            return carry

        lax.fori_loop(0, NCHUNK, body, 0)

    return k(*ids_list, sp_t, mv_t, it_t, ab_t)


def _tc_body(st_ids_ref, we_ids_ref, te_ids_ref, po_ids_ref,
             sp_ref, mv_ref, it_ref, ab_ref, hp_ref, bo_ref,
             st_t_ref, we_t_ref, te_t_ref, po_t_ref,
             hp_W_ref, hp_b_ref, boost_W_ref, boost_b_ref,
             wproj_ref, proj_b_ref, gamma_ref, beta_ref, out_ref):
    f32 = jnp.float32

    def onehot_emb(ids_ref, tbl_ref, vocab):
        ids = ids_ref[0, 0, :]
        oh = (ids[:, None] == lax.broadcasted_iota(
            jnp.int32, (TOK_TILE, vocab), 1)).astype(f32)
        return jnp.dot(oh, tbl_ref[...], preferred_element_type=f32)

    st_emb = onehot_emb(st_ids_ref, st_t_ref, 8)
    we_emb = onehot_emb(we_ids_ref, we_t_ref, 10)
    te_emb = onehot_emb(te_ids_ref, te_t_ref, 5)
    po_emb = onehot_emb(po_ids_ref, po_t_ref, 12)
    hp_emb = hp_ref[...] * hp_W_ref[...] + hp_b_ref[...][None, :]
    bo_emb = jnp.dot(bo_ref[...], boost_W_ref[...],
                     preferred_element_type=f32) + boost_b_ref[...][None, :]

    bf16 = jnp.bfloat16
    combined = jnp.concatenate([
        sp_ref[...], mv_ref[...], it_ref[:, :64], ab_ref[:, :64],
        hp_emb, bo_emb, st_emb, we_emb, te_emb, po_emb], axis=1).astype(bf16)

    acc = jnp.dot(combined, wproj_ref[...], preferred_element_type=f32)
    acc = acc + proj_b_ref[...][None, :]
    mean = jnp.mean(acc, axis=1, keepdims=True)
    cen = acc - mean
    var = jnp.mean(cen * cen, axis=1, keepdims=True)
    y = cen * lax.rsqrt(var + EPS)
    out_ref[...] = y * gamma_ref[...][None, :] + beta_ref[...][None, :]


def _full(shape):
    nd = len(shape)
    return pl.BlockSpec(shape, lambda i: (0,) * nd)


def kernel(species_ids, move_ids, item_ids, ability_ids, hp_values, stat_boosts,
           status_ids, weather_ids, terrain_ids, position_ids,
           species_table, move_table, item_table, ability_table,
           hp_W, hp_b, boost_W, boost_b,
           status_table, weather_table, terrain_table, position_table,
           proj_W, proj_b, ln_gamma, ln_beta):
    i32 = jnp.int32
    ids_list = [a.T.reshape(NW, TOK_PER_W).astype(i32)
                for a in (species_ids, move_ids, item_ids, ability_ids)]

    pad64 = lambda t: jnp.pad(t, ((0, 0), (0, 64)))
    sp_e, mv_e, it_e, ab_e = _sc_gather(
        ids_list, species_table, move_table, pad64(item_table),
        pad64(ability_table))

    st3 = status_ids.T.reshape(GRID, 1, TOK_TILE).astype(i32)
    we3 = weather_ids.T.reshape(GRID, 1, TOK_TILE).astype(i32)
    te3 = terrain_ids.T.reshape(GRID, 1, TOK_TILE).astype(i32)
    po3 = position_ids.T.reshape(GRID, 1, TOK_TILE).astype(i32)
    hp2 = hp_values.T.reshape(N, 1)
    bo2 = stat_boosts.transpose(1, 0, 2).reshape(N, 7)
    wproj_bf = proj_W.astype(jnp.bfloat16)

    ids_spec = pl.BlockSpec((1, 1, TOK_TILE), lambda i: (i, 0, 0))

    out = pl.pallas_call(
        _tc_body,
        grid=(GRID,),
        in_specs=[
            ids_spec, ids_spec, ids_spec, ids_spec,
            pl.BlockSpec((TOK_TILE, 256), lambda i: (i, 0)),
            pl.BlockSpec((TOK_TILE, 128), lambda i: (i, 0)),
            pl.BlockSpec((TOK_TILE, 128), lambda i: (i, 0)),
            pl.BlockSpec((TOK_TILE, 128), lambda i: (i, 0)),
            pl.BlockSpec((TOK_TILE, 1), lambda i: (i, 0)),
            pl.BlockSpec((TOK_TILE, 7), lambda i: (i, 0)),
            _full((8, 32)), _full((10, 32)), _full((5, 32)), _full((12, 64)),
            _full((1, 32)), _full((32,)), _full((7, 32)), _full((32,)),
            _full((736, HIDDEN)), _full((HIDDEN,)),
            _full((HIDDEN,)), _full((HIDDEN,)),
        ],
        out_specs=pl.BlockSpec((TOK_TILE, HIDDEN), lambda i: (i, 0)),
        out_shape=jax.ShapeDtypeStruct((N, HIDDEN), jnp.float32),
        compiler_params=pltpu.CompilerParams(
            dimension_semantics=("arbitrary",)),
    )(st3, we3, te3, po3, sp_e, mv_e, it_e, ab_e, hp2, bo2,
      status_table, weather_table, terrain_table, position_table,
      hp_W, hp_b, boost_W, boost_b,
      wproj_bf, proj_b, ln_gamma, ln_beta)

    return out.reshape(S, B, HIDDEN).transpose(1, 0, 2)
